# baseline (speedup 1.0000x reference)
import functools

import jax
import jax.numpy as jnp
from jax import lax
from jax.experimental import pallas as pl
from jax.experimental.pallas import tpu as pltpu
from jax.experimental.pallas import tpu_sc as plsc

NUM_ROWS = 4096
DIM = 6
LANES = 16
NUM_CORES = 2
NUM_SUBCORES = 16
NUM_WORKERS = NUM_CORES * NUM_SUBCORES
ROWS_PER_W = NUM_ROWS // NUM_WORKERS
FLOATS_PER_W = ROWS_PER_W * DIM
VECS_PER_W = FLOATS_PER_W // LANES


@functools.partial(
    pl.kernel,
    out_type=jax.ShapeDtypeStruct((NUM_ROWS * DIM,), jnp.float32),
    mesh=plsc.VectorSubcoreMesh(core_axis_name="c", subcore_axis_name="s"),
    scratch_types=[
        pltpu.VMEM((ROWS_PER_W,), jnp.int32),
        pltpu.VMEM((LANES,), jnp.float32),
        pltpu.VMEM((FLOATS_PER_W,), jnp.float32),
    ],
    compiler_params=pltpu.CompilerParams(needs_layout_passes=False),
)
def _lookup(idx_hbm, tbl_hbm, out_hbm, idx_v, tbl_v, buf_v):
    wid = lax.axis_index("s") * NUM_CORES + lax.axis_index("c")
    pltpu.sync_copy(idx_hbm.at[pl.ds(wid * ROWS_PER_W, ROWS_PER_W)], idx_v)
    pltpu.sync_copy(tbl_hbm, tbl_v)
    lane = lax.iota(jnp.int32, LANES)
    for k in range(VECS_PER_W):
        pos = lane + (k * LANES)
        row = pos // DIM
        col = pos - row * DIM
        idx = plsc.load_gather(idx_v, [row])
        idx = jnp.clip(idx, 0, 0)
        vals = plsc.load_gather(tbl_v, [idx * DIM + col])
        buf_v[pl.ds(k * LANES, LANES)] = vals
    pltpu.sync_copy(buf_v, out_hbm.at[pl.ds(wid * FLOATS_PER_W, FLOATS_PER_W)])


def kernel(indices, table):
    idx = indices.astype(jnp.int32)
    tbl = jnp.pad(table.reshape(DIM), (0, LANES - DIM))
    out = _lookup(idx, tbl)
    return out.reshape(NUM_ROWS, DIM)


# speedup vs baseline: 1.1023x; 1.1023x over previous
import functools

import jax
import jax.numpy as jnp
from jax import lax
from jax.experimental import pallas as pl
from jax.experimental.pallas import tpu as pltpu
from jax.experimental.pallas import tpu_sc as plsc

NUM_ROWS = 4096
DIM = 6
LANES = 16
NUM_CORES = 1
NUM_SUBCORES = 16
NUM_WORKERS = NUM_CORES * NUM_SUBCORES
ROWS_PER_W = NUM_ROWS // NUM_WORKERS
FLOATS_PER_W = ROWS_PER_W * DIM
VECS_PER_W = FLOATS_PER_W // LANES


@functools.partial(
    pl.kernel,
    out_type=jax.ShapeDtypeStruct((NUM_ROWS, DIM), jnp.float32),
    mesh=plsc.VectorSubcoreMesh(
        core_axis_name="c", subcore_axis_name="s", num_cores=NUM_CORES
    ),
    scratch_types=[
        pltpu.VMEM((ROWS_PER_W,), jnp.int32),
        pltpu.VMEM((1, DIM), jnp.float32),
        pltpu.VMEM((ROWS_PER_W, DIM), jnp.float32),
        pltpu.SemaphoreType.DMA,
        pltpu.SemaphoreType.DMA,
    ],
    compiler_params=pltpu.CompilerParams(
        needs_layout_passes=False, disable_bounds_checks=True,
        skip_device_barrier=True
    ),
)
def _lookup(idx_hbm, tbl_hbm, out_hbm, idx_v, tbl_v, buf_v, sem1, sem2):
    wid = lax.axis_index("s") * NUM_CORES + lax.axis_index("c")
    cp1 = pltpu.make_async_copy(
        idx_hbm.at[pl.ds(wid * ROWS_PER_W, ROWS_PER_W)], idx_v, sem1
    )
    cp2 = pltpu.make_async_copy(tbl_hbm, tbl_v, sem2)
    cp1.start()
    cp2.start()
    cp1.wait()
    cp2.wait()
    lane = lax.iota(jnp.int32, LANES)

    def body(k, carry):
        pos = lane + k * LANES
        row = pos // DIM
        col = pos - row * DIM
        idx = plsc.load_gather(idx_v, [row])
        idx = jnp.clip(idx, 0, 0)
        vals = plsc.load_gather(tbl_v, [idx, col])
        plsc.store_scatter(buf_v, [row, col], vals)
        return carry

    lax.fori_loop(0, VECS_PER_W, body, 0)
    pltpu.sync_copy(buf_v, out_hbm.at[pl.ds(wid * ROWS_PER_W, ROWS_PER_W)])


def kernel(indices, table):
    return _lookup(indices.astype(jnp.int32), table)
